# baseline (speedup 1.0000x reference)
import functools

import jax
import jax.numpy as jnp
import numpy as np
from jax import lax
from jax.experimental import pallas as pl
from jax.experimental.pallas import tpu as pltpu
from jax.experimental.pallas import tpu_sc as plsc

NC = 2
NS = 16
NW = NC * NS
CH = 128
NBUF = 2


def _sc_embedding_bag(B, G, D, center_idx, ngram_idx, slot_idx, W_in, W_sub):
    b_per_w = B // NW
    n_ctr = b_per_w // CH
    n_sub = (b_per_w * G) // CH
    n_sub_p = n_sub + NBUF

    mesh = plsc.VectorSubcoreMesh(core_axis_name="c", subcore_axis_name="s")

    @functools.partial(
        pl.kernel,
        mesh=mesh,
        out_type=jax.ShapeDtypeStruct((B, D), jnp.float32),
        compiler_params=pltpu.CompilerParams(use_tc_tiling_on_sc=False),
        scratch_types=dict(
            ctr_v=pltpu.VMEM((n_ctr, CH), jnp.int32),
            ng_v=pltpu.VMEM((n_sub_p, CH), jnp.int32),
            slot_v=pltpu.VMEM((n_sub_p, CH), jnp.int32),
            bufs=pltpu.VMEM((NBUF, CH, D), jnp.float32),
            ctrbuf=pltpu.VMEM((CH, D), jnp.float32),
            outbuf=pltpu.VMEM((b_per_w, D), jnp.float32),
            acc=pltpu.VMEM_SHARED((NS * b_per_w + 8, D), jnp.float32),
            gsems=pltpu.SemaphoreType.DMA((NBUF,)),
            sem=pltpu.SemaphoreType.DMA,
        ),
    )
    def k(ctr_hbm, ng_hbm, slot_hbm, w_in, w_sub, out,
          ctr_v, ng_v, slot_v, bufs, ctrbuf, outbuf, acc, gsems, sem):
        cid = lax.axis_index("c")
        sid = lax.axis_index("s")
        wid = sid * NC + cid

        pltpu.sync_copy(ctr_hbm.at[wid], ctr_v)
        pltpu.sync_copy(ng_hbm.at[wid], ng_v)
        pltpu.sync_copy(slot_hbm.at[sid], slot_v)

        for c in range(n_ctr):
            pltpu.async_copy(w_in.at[ctr_v.at[c]], ctrbuf, sem).wait()
            pltpu.sync_copy(
                ctrbuf, acc.at[pl.ds(sid * b_per_w + c * CH, CH)])

        for b in range(NBUF):
            pltpu.async_copy(w_sub.at[ng_v.at[b]], bufs.at[b], gsems.at[b])

        def step(i, carry):
            c = i * NBUF
            for b in range(NBUF):
                pltpu.make_async_copy(
                    w_sub.at[ng_v.at[c + b]], bufs.at[b], gsems.at[b]).wait()
                pltpu.sync_copy(bufs.at[b], acc.at[slot_v.at[c + b]], add=True)
                pltpu.async_copy(
                    w_sub.at[ng_v.at[c + b + NBUF]], bufs.at[b], gsems.at[b])
            return carry

        lax.fori_loop(0, n_sub // NBUF, step, 0)
        for b in range(NBUF):
            pltpu.make_async_copy(
                w_sub.at[ng_v.at[n_sub + b]], bufs.at[b], gsems.at[b]).wait()

        pltpu.sync_copy(acc.at[pl.ds(sid * b_per_w, b_per_w)], outbuf)
        pltpu.sync_copy(outbuf, out.at[pl.ds(wid * b_per_w, b_per_w)])

    return k(center_idx, ngram_idx, slot_idx, W_in, W_sub)


def kernel(center_ids, ngram_ids, W_in, W_sub):
    B, G = ngram_ids.shape
    D = W_in.shape[1]
    b_per_w = B // NW
    n_sub = (b_per_w * G) // CH
    n_sub_p = n_sub + NBUF

    ctr = center_ids.astype(jnp.int32).reshape(NW, b_per_w // CH, CH)
    ng = ngram_ids.astype(jnp.int32).reshape(NW, b_per_w * G)
    pad = jnp.zeros((NW, NBUF * CH), jnp.int32)
    ng = jnp.concatenate([ng, pad], axis=1).reshape(NW, n_sub_p, CH)

    slots = np.repeat(np.arange(b_per_w, dtype=np.int32), G)
    slots = np.concatenate(
        [slots, np.zeros(NBUF * CH, np.int32)]).reshape(n_sub_p, CH)
    slot_np = np.empty((NS, n_sub_p, CH), np.int32)
    for s in range(NS):
        slot_np[s] = slots + s * b_per_w
    slot_np[:, n_sub:, :] = NS * b_per_w
    slot_idx = jnp.asarray(slot_np)

    return _sc_embedding_bag(B, G, D, ctr, ng, slot_idx, W_in, W_sub)
